# baseline (speedup 1.0000x reference)
import functools

import jax
import jax.numpy as jnp
from jax import lax
from jax.experimental import pallas as pl
from jax.experimental.pallas import tpu as pltpu
from jax.experimental.pallas import tpu_sc as plsc

N = 32768
D = 256
H = 128
M = 16
BLK = 8192
HALF = N // 2

NS = plsc.get_sparse_core_info().num_subcores
L = 16
CHUNK = HALF // NS


def _mlp_body(x_ref, w1_ref, b1_ref, w2_ref, b2_ref, y_ref):
    x = x_ref[...].astype(jnp.bfloat16)
    h = jnp.dot(x, w1_ref[...].astype(jnp.bfloat16),
                preferred_element_type=jnp.float32)
    h = h + b1_ref[...]
    h = h * jax.nn.sigmoid(h)
    y = jnp.dot(h, w2_ref[...], preferred_element_type=jnp.float32)
    y_ref[...] = y + b2_ref[...]


def _mlp_half(x, w1, b1, w2, b2, phase):
    nblk = HALF // BLK
    return pl.pallas_call(
        _mlp_body,
        grid=(nblk,),
        in_specs=[
            pl.BlockSpec((BLK, D), lambda i, p=phase: (p * nblk + i, 0)),
            pl.BlockSpec((D, H), lambda i: (0, 0)),
            pl.BlockSpec((1, H), lambda i: (0, 0)),
            pl.BlockSpec((H, 1), lambda i: (0, 0)),
            pl.BlockSpec((1, 1), lambda i: (0, 0)),
        ],
        out_specs=pl.BlockSpec((BLK, 1), lambda i: (i, 0)),
        out_shape=jax.ShapeDtypeStruct((HALF, 1), jnp.float32),
        compiler_params=pltpu.CompilerParams(skip_device_barrier=True),
    )(x, w1, b1.reshape(1, H), w2, b2.reshape(1, 1))


def _make_sc_segsum(off):
    @functools.partial(
        pl.kernel,
        mesh=plsc.VectorSubcoreMesh(core_axis_name="c", subcore_axis_name="s",
                                    num_cores=1),
        out_type=jax.ShapeDtypeStruct((M,), jnp.float32),
        scratch_types=[
            pltpu.VMEM((CHUNK,), jnp.float32),
            pltpu.VMEM((CHUNK,), jnp.int32),
            pltpu.VMEM((128,), jnp.float32),
            pltpu.VMEM((128,), jnp.float32),
            pltpu.VMEM((NS, 128), jnp.float32),
            pltpu.VMEM_SHARED((NS, 128), jnp.float32),
            pltpu.SemaphoreType.DMA,
            pltpu.SemaphoreType.DMA,
        ],
        compiler_params=pltpu.CompilerParams(needs_layout_passes=False,
                                             skip_device_barrier=True),
    )
    def _sc_segsum(y_hbm, idx_hbm, out_hbm, y_v, idx_v, acc_v, acc2_v, red_v,
                   shared, sem1, sem2):
        s = lax.axis_index("s")
        base = s * CHUNK
        cp1 = pltpu.async_copy(y_hbm.at[pl.ds(base, CHUNK)], y_v, sem1)
        cp2 = pltpu.async_copy(idx_hbm.at[pl.ds(off + base, CHUNK)], idx_v,
                               sem2)
        for j in range(128 // L):
            acc_v[pl.ds(j * L, L)] = jnp.zeros((L,), jnp.float32)
            acc2_v[pl.ds(j * L, L)] = jnp.zeros((L,), jnp.float32)
        cp1.wait()
        cp2.wait()

        def body(j, carry):
            o = 2 * j * L
            plsc.addupdate_scatter(acc_v, [idx_v[pl.ds(o, L)]],
                                   y_v[pl.ds(o, L)])
            plsc.addupdate_scatter(acc2_v, [idx_v[pl.ds(o + L, L)]],
                                   y_v[pl.ds(o + L, L)])
            return carry

        lax.fori_loop(0, CHUNK // (2 * L), body, 0, unroll=8)
        acc_v[pl.ds(0, M)] = acc_v[pl.ds(0, M)] + acc2_v[pl.ds(0, M)]

        pltpu.sync_copy(acc_v, shared.at[s])
        plsc.subcore_barrier()

        @pl.when(s == 0)
        def _reduce():
            pltpu.sync_copy(shared, red_v)
            total = red_v[0, pl.ds(0, M)]
            for j in range(1, NS):
                total = total + red_v[j, pl.ds(0, M)]
            acc_v[pl.ds(0, M)] = total
            pltpu.sync_copy(acc_v.at[pl.ds(0, M)], out_hbm)

    return _sc_segsum


_sc_segsum0 = _make_sc_segsum(0)
_sc_segsum1 = _make_sc_segsum(HALF)


def kernel(scalar_representation, idx_m, W1, b1, W2, b2):
    idx32 = idx_m.astype(jnp.int32)
    y0 = _mlp_half(scalar_representation, W1, b1, W2, b2, 0)
    p0 = _sc_segsum0(y0.reshape(HALF), idx32)
    y1 = _mlp_half(scalar_representation, W1, b1, W2, b2, 1)
    p1 = _sc_segsum1(y1.reshape(HALF), idx32)
    return p0 + p1


# speedup vs baseline: 1.0772x; 1.0772x over previous
import functools

import jax
import jax.numpy as jnp
from jax import lax
from jax.experimental import pallas as pl
from jax.experimental.pallas import tpu as pltpu
from jax.experimental.pallas import tpu_sc as plsc

N = 32768
D = 256
H = 128
M = 16
BLK = 8192

NS = plsc.get_sparse_core_info().num_subcores
L = 16


def _mlp_body(x_ref, w1_ref, b1_ref, w2_ref, b2_ref, y_ref):
    x = x_ref[...].astype(jnp.bfloat16)
    h = jnp.dot(x, w1_ref[...].astype(jnp.bfloat16),
                preferred_element_type=jnp.float32)
    h = h + b1_ref[...]
    h = h * jax.nn.sigmoid(h)
    y = jnp.dot(h, w2_ref[...], preferred_element_type=jnp.float32)
    y_ref[...] = y + b2_ref[...]


def _mlp(x, w1, b1, w2, b2):
    return pl.pallas_call(
        _mlp_body,
        grid=(N // BLK,),
        in_specs=[
            pl.BlockSpec((BLK, D), lambda i: (i, 0)),
            pl.BlockSpec((D, H), lambda i: (0, 0)),
            pl.BlockSpec((1, H), lambda i: (0, 0)),
            pl.BlockSpec((H, 1), lambda i: (0, 0)),
            pl.BlockSpec((1, 1), lambda i: (0, 0)),
        ],
        out_specs=pl.BlockSpec((BLK, 1), lambda i: (i, 0)),
        out_shape=jax.ShapeDtypeStruct((N, 1), jnp.float32),
        compiler_params=pltpu.CompilerParams(skip_device_barrier=True),
    )(x, w1, b1.reshape(1, H), w2, b2.reshape(1, 1))


CHUNK1 = N // NS


@functools.partial(
    pl.kernel,
    mesh=plsc.VectorSubcoreMesh(core_axis_name="c", subcore_axis_name="s",
                                num_cores=1),
    out_type=jax.ShapeDtypeStruct((M,), jnp.float32),
    scratch_types=[
        pltpu.VMEM((CHUNK1,), jnp.float32),
        pltpu.VMEM((CHUNK1,), jnp.int32),
        pltpu.VMEM((128,), jnp.float32),
        pltpu.VMEM((128,), jnp.float32),
        pltpu.VMEM((NS, 128), jnp.float32),
        pltpu.VMEM_SHARED((NS, 128), jnp.float32),
        pltpu.SemaphoreType.DMA,
        pltpu.SemaphoreType.DMA,
    ],
    compiler_params=pltpu.CompilerParams(needs_layout_passes=False,
                                         skip_device_barrier=True),
)
def _sc_segsum(y_hbm, idx_hbm, out_hbm, y_v, idx_v, acc_v, acc2_v, red_v,
               shared, sem1, sem2):
    s = lax.axis_index("s")
    base = s * CHUNK1
    cp1 = pltpu.async_copy(y_hbm.at[pl.ds(base, CHUNK1)], y_v, sem1)
    cp2 = pltpu.async_copy(idx_hbm.at[pl.ds(base, CHUNK1)], idx_v, sem2)
    for j in range(128 // L):
        acc_v[pl.ds(j * L, L)] = jnp.zeros((L,), jnp.float32)
        acc2_v[pl.ds(j * L, L)] = jnp.zeros((L,), jnp.float32)
    cp1.wait()
    cp2.wait()

    def body(j, carry):
        off = 2 * j * L
        plsc.addupdate_scatter(acc_v, [idx_v[pl.ds(off, L)]],
                               y_v[pl.ds(off, L)])
        plsc.addupdate_scatter(acc2_v, [idx_v[pl.ds(off + L, L)]],
                               y_v[pl.ds(off + L, L)])
        return carry

    lax.fori_loop(0, CHUNK1 // (2 * L), body, 0, unroll=8)
    acc_v[pl.ds(0, M)] = acc_v[pl.ds(0, M)] + acc2_v[pl.ds(0, M)]

    pltpu.sync_copy(acc_v, shared.at[s])
    plsc.subcore_barrier()

    @pl.when(s == 0)
    def _reduce():
        pltpu.sync_copy(shared, red_v)
        total = red_v[0, pl.ds(0, M)]
        for j in range(1, NS):
            total = total + red_v[j, pl.ds(0, M)]
        acc_v[pl.ds(0, M)] = total
        pltpu.sync_copy(acc_v.at[pl.ds(0, M)], out_hbm)


def kernel(scalar_representation, idx_m, W1, b1, W2, b2):
    y = _mlp(scalar_representation, W1, b1, W2, b2)
    return _sc_segsum(y.reshape(N), idx_m.astype(jnp.int32))


# speedup vs baseline: 1.0780x; 1.0008x over previous
import functools

import jax
import jax.numpy as jnp
from jax import lax
from jax.experimental import pallas as pl
from jax.experimental.pallas import tpu as pltpu
from jax.experimental.pallas import tpu_sc as plsc

N = 32768
D = 256
H = 128
M = 16
BLK = 8192

NS = 16
L = 16


def _mlp_body(x_ref, w1_ref, b1_ref, w2_ref, b2_ref, y_ref):
    x = x_ref[...].astype(jnp.bfloat16)
    h = jnp.dot(x, w1_ref[...].astype(jnp.bfloat16),
                preferred_element_type=jnp.float32)
    h = h + b1_ref[...]
    h = h * jax.nn.sigmoid(h)
    y = jnp.dot(h, w2_ref[...], preferred_element_type=jnp.float32)
    y_ref[...] = y + b2_ref[...]


def _mlp(x, w1, b1, w2, b2):
    return pl.pallas_call(
        _mlp_body,
        grid=(N // BLK,),
        in_specs=[
            pl.BlockSpec((BLK, D), lambda i: (i, 0)),
            pl.BlockSpec((D, H), lambda i: (0, 0)),
            pl.BlockSpec((1, H), lambda i: (0, 0)),
            pl.BlockSpec((H, 1), lambda i: (0, 0)),
            pl.BlockSpec((1, 1), lambda i: (0, 0)),
        ],
        out_specs=pl.BlockSpec((BLK, 1), lambda i: (i, 0)),
        out_shape=jax.ShapeDtypeStruct((N, 1), jnp.float32),
        compiler_params=pltpu.CompilerParams(skip_device_barrier=True),
    )(x, w1, b1.reshape(1, H), w2, b2.reshape(1, 1))


CHUNK1 = N // NS


@functools.partial(
    pl.kernel,
    mesh=plsc.VectorSubcoreMesh(core_axis_name="c", subcore_axis_name="s",
                                num_cores=1),
    out_type=jax.ShapeDtypeStruct((M,), jnp.float32),
    scratch_types=[
        pltpu.VMEM((CHUNK1,), jnp.float32),
        pltpu.VMEM((CHUNK1,), jnp.int32),
        pltpu.VMEM((128,), jnp.float32),
        pltpu.VMEM((128,), jnp.float32),
        pltpu.VMEM((NS, 128), jnp.float32),
        pltpu.VMEM_SHARED((NS, 128), jnp.float32),
        pltpu.SemaphoreType.DMA,
        pltpu.SemaphoreType.DMA,
    ],
    compiler_params=pltpu.CompilerParams(needs_layout_passes=False,
                                         skip_device_barrier=True),
)
def _sc_segsum(y_hbm, idx_hbm, out_hbm, y_v, idx_v, acc_v, acc2_v, red_v,
               shared, sem1, sem2):
    s = lax.axis_index("s")
    base = s * CHUNK1
    cp1 = pltpu.async_copy(y_hbm.at[pl.ds(base, CHUNK1)], y_v, sem1)
    cp2 = pltpu.async_copy(idx_hbm.at[pl.ds(base, CHUNK1)], idx_v, sem2)
    for j in range(128 // L):
        acc_v[pl.ds(j * L, L)] = jnp.zeros((L,), jnp.float32)
        acc2_v[pl.ds(j * L, L)] = jnp.zeros((L,), jnp.float32)
    cp1.wait()
    cp2.wait()

    def body(j, carry):
        off = 2 * j * L
        plsc.addupdate_scatter(acc_v, [idx_v[pl.ds(off, L)]],
                               y_v[pl.ds(off, L)])
        plsc.addupdate_scatter(acc2_v, [idx_v[pl.ds(off + L, L)]],
                               y_v[pl.ds(off + L, L)])
        return carry

    lax.fori_loop(0, CHUNK1 // (2 * L), body, 0, unroll=8)
    acc_v[pl.ds(0, M)] = acc_v[pl.ds(0, M)] + acc2_v[pl.ds(0, M)]

    pltpu.sync_copy(acc_v, shared.at[s])
    plsc.subcore_barrier()

    @pl.when(s == 0)
    def _reduce():
        pltpu.sync_copy(shared, red_v)
        total = red_v[0, pl.ds(0, M)]
        for j in range(1, NS):
            total = total + red_v[j, pl.ds(0, M)]
        acc_v[pl.ds(0, M)] = total
        pltpu.sync_copy(acc_v.at[pl.ds(0, M)], out_hbm)


def kernel(scalar_representation, idx_m, W1, b1, W2, b2):
    y = _mlp(scalar_representation, W1, b1, W2, b2)
    return _sc_segsum(y.reshape(N), idx_m.astype(jnp.int32))


# speedup vs baseline: 1.0858x; 1.0072x over previous
import functools

import jax
import jax.numpy as jnp
from jax import lax
from jax.experimental import pallas as pl
from jax.experimental.pallas import tpu as pltpu
from jax.experimental.pallas import tpu_sc as plsc

N = 32768
D = 256
H = 128
M = 16
BLK = 16384

NS = 16
L = 16


def _mlp_body(x_ref, w1_ref, b1_ref, w2_ref, b2_ref, y_ref):
    x = x_ref[...].astype(jnp.bfloat16)
    h = jnp.dot(x, w1_ref[...].astype(jnp.bfloat16),
                preferred_element_type=jnp.float32)
    h = h + b1_ref[...]
    h = h * jax.nn.sigmoid(h)
    y = jnp.dot(h, w2_ref[...], preferred_element_type=jnp.float32)
    y_ref[...] = y + b2_ref[...]


def _mlp(x, w1, b1, w2, b2):
    return pl.pallas_call(
        _mlp_body,
        grid=(N // BLK,),
        in_specs=[
            pl.BlockSpec((BLK, D), lambda i: (i, 0)),
            pl.BlockSpec((D, H), lambda i: (0, 0)),
            pl.BlockSpec((1, H), lambda i: (0, 0)),
            pl.BlockSpec((H, 1), lambda i: (0, 0)),
            pl.BlockSpec((1, 1), lambda i: (0, 0)),
        ],
        out_specs=pl.BlockSpec((BLK, 1), lambda i: (i, 0)),
        out_shape=jax.ShapeDtypeStruct((N, 1), jnp.float32),
        compiler_params=pltpu.CompilerParams(skip_device_barrier=True, vmem_limit_bytes=67108864),
    )(x, w1, b1.reshape(1, H), w2, b2.reshape(1, 1))


CHUNK1 = N // NS


@functools.partial(
    pl.kernel,
    mesh=plsc.VectorSubcoreMesh(core_axis_name="c", subcore_axis_name="s",
                                num_cores=1),
    out_type=jax.ShapeDtypeStruct((M,), jnp.float32),
    scratch_types=[
        pltpu.VMEM((CHUNK1,), jnp.float32),
        pltpu.VMEM((CHUNK1,), jnp.int32),
        pltpu.VMEM((128,), jnp.float32),
        pltpu.VMEM((128,), jnp.float32),
        pltpu.VMEM((NS, 128), jnp.float32),
        pltpu.VMEM_SHARED((NS, 128), jnp.float32),
        pltpu.SemaphoreType.DMA,
        pltpu.SemaphoreType.DMA,
    ],
    compiler_params=pltpu.CompilerParams(needs_layout_passes=False,
                                         skip_device_barrier=True),
)
def _sc_segsum(y_hbm, idx_hbm, out_hbm, y_v, idx_v, acc_v, acc2_v, red_v,
               shared, sem1, sem2):
    s = lax.axis_index("s")
    base = s * CHUNK1
    cp1 = pltpu.async_copy(y_hbm.at[pl.ds(base, CHUNK1)], y_v, sem1)
    cp2 = pltpu.async_copy(idx_hbm.at[pl.ds(base, CHUNK1)], idx_v, sem2)
    for j in range(128 // L):
        acc_v[pl.ds(j * L, L)] = jnp.zeros((L,), jnp.float32)
        acc2_v[pl.ds(j * L, L)] = jnp.zeros((L,), jnp.float32)
    cp1.wait()
    cp2.wait()

    def body(j, carry):
        off = 2 * j * L
        plsc.addupdate_scatter(acc_v, [idx_v[pl.ds(off, L)]],
                               y_v[pl.ds(off, L)])
        plsc.addupdate_scatter(acc2_v, [idx_v[pl.ds(off + L, L)]],
                               y_v[pl.ds(off + L, L)])
        return carry

    lax.fori_loop(0, CHUNK1 // (2 * L), body, 0, unroll=8)
    acc_v[pl.ds(0, M)] = acc_v[pl.ds(0, M)] + acc2_v[pl.ds(0, M)]

    pltpu.sync_copy(acc_v, shared.at[s])
    plsc.subcore_barrier()

    @pl.when(s == 0)
    def _reduce():
        pltpu.sync_copy(shared, red_v)
        total = red_v[0, pl.ds(0, M)]
        for j in range(1, NS):
            total = total + red_v[j, pl.ds(0, M)]
        acc_v[pl.ds(0, M)] = total
        pltpu.sync_copy(acc_v.at[pl.ds(0, M)], out_hbm)


def kernel(scalar_representation, idx_m, W1, b1, W2, b2):
    y = _mlp(scalar_representation, W1, b1, W2, b2)
    return _sc_segsum(y.reshape(N), idx_m.astype(jnp.int32))
